# baseline (speedup 1.0000x reference)
import functools

import jax
import jax.numpy as jnp
from jax import lax
from jax.experimental import pallas as pl
from jax.experimental.pallas import tpu as pltpu
from jax.experimental.pallas import tpu_sc as plsc


N = 10000
E = 320000
D = 128
G = 256

NPAD = 10240
NW = 32
CHUNK = 128
NCHUNK = 80
EPW = CHUNK * NCHUNK
EPAD = NW * EPW
RPT = NPAD // 16
L = 16


def _mm_kernel(x_ref, w_ref, o_ref):
    o_ref[...] = jnp.dot(x_ref[...], w_ref[...],
                         preferred_element_type=jnp.float32)


def _mm(x, w, block=1024):
    m, k = x.shape
    n = w.shape[1]
    return pl.pallas_call(
        _mm_kernel,
        grid=(m // block,),
        in_specs=[
            pl.BlockSpec((block, k), lambda i: (i, 0)),
            pl.BlockSpec((k, n), lambda i: (0, 0)),
        ],
        out_specs=pl.BlockSpec((block, n), lambda i: (i, 0)),
        out_shape=jax.ShapeDtypeStruct((m, n), jnp.float32),
    )(x, w)


def _zero_1d(ref):
    z = jnp.zeros((L,), jnp.float32)

    def body(i, _):
        ref[pl.ds(i * L, L)] = z
        return 0

    lax.fori_loop(0, ref.shape[0] // L, body, 0)


def _zero_rows(ref):
    z = jnp.zeros((L,), jnp.float32)

    def body(r, _):
        for j in range(D // L):
            ref[r, pl.ds(j * L, L)] = z
        return 0

    lax.fori_loop(0, ref.shape[0], body, 0)


_MESH = plsc.VectorSubcoreMesh(core_axis_name="c", subcore_axis_name="s")
_SC_PARAMS = pltpu.CompilerParams(needs_layout_passes=False)


def _p0_body(dst_hbm, e1_hbm, e2_hbm, cnt_out, s1_out, s2_out,
             dst_v, e1_v, e2_v, cnt_v, s1_v, s2_v):
    cidx = lax.axis_index("c")
    sidx = lax.axis_index("s")
    wid = cidx * 16 + sidx
    pltpu.sync_copy(dst_hbm.at[wid], dst_v)
    pltpu.sync_copy(e1_hbm.at[wid], e1_v)
    pltpu.sync_copy(e2_hbm.at[wid], e2_v)
    _zero_1d(cnt_v)
    _zero_1d(s1_v)
    _zero_1d(s2_v)
    ones = jnp.ones((L,), jnp.float32)

    def body(c, _):
        for k in range(CHUNK // L):
            didx = dst_v[c, pl.ds(k * L, L)]
            plsc.addupdate_scatter(cnt_v, [didx], ones)
            plsc.addupdate_scatter(s1_v, [didx], e1_v[c, pl.ds(k * L, L)])
            plsc.addupdate_scatter(s2_v, [didx], e2_v[c, pl.ds(k * L, L)])
        return 0

    lax.fori_loop(0, NCHUNK, body, 0)
    pltpu.sync_copy(cnt_v, cnt_out.at[wid])
    pltpu.sync_copy(s1_v, s1_out.at[wid])
    pltpu.sync_copy(s2_v, s2_out.at[wid])


_p0_call = pl.kernel(
    _p0_body,
    out_type=[jax.ShapeDtypeStruct((NW, NPAD), jnp.float32)] * 3,
    mesh=_MESH,
    compiler_params=_SC_PARAMS,
    scratch_types=[
        pltpu.VMEM((NCHUNK, CHUNK), jnp.int32),
        pltpu.VMEM((NCHUNK, CHUNK), jnp.float32),
        pltpu.VMEM((NCHUNK, CHUNK), jnp.float32),
        pltpu.VMEM((NPAD,), jnp.float32),
        pltpu.VMEM((NPAD,), jnp.float32),
        pltpu.VMEM((NPAD,), jnp.float32),
    ],
)


def _layer_body(gat, *refs):
    if gat:
        (xw_hbm, pack_hbm, al_hbm, ar_hbm,
         num_out, den_out,
         pb0, pb1, al_v, ar_v, den_v, w_v, rows_v, num_sh,
         sg, sp0, sp1) = refs
    else:
        (xw_hbm, pack_hbm, al_hbm,
         num_out,
         pb0, pb1, al_v, w_v, rows_v, num_sh,
         sg, sp0, sp1) = refs
    cidx = lax.axis_index("c")
    sidx = lax.axis_index("s")
    wid = cidx * 16 + sidx
    pbuf = (pb0, pb1)
    sp = (sp0, sp1)

    pltpu.sync_copy(al_hbm, al_v)
    if gat:
        pltpu.sync_copy(ar_hbm, ar_v)
        _zero_1d(den_v)

    _zero_rows(rows_v)
    for i in range(RPT // CHUNK):
        pltpu.sync_copy(rows_v, num_sh.at[pl.ds(sidx * RPT + i * CHUNK, CHUNK)])
    plsc.subcore_barrier()

    pltpu.sync_copy(pack_hbm.at[wid, 0], pb0)

    def block(t, b):
        c = 2 * t + b
        pb = pbuf[b]
        gdesc = pltpu.async_copy(xw_hbm.at[pb.at[0]], rows_v, sg)
        pltpu.async_copy(pack_hbm.at[wid, c + 1], pbuf[b ^ 1], sp[b ^ 1])
        for k in range(CHUNK // L):
            sl = pl.ds(k * L, L)
            s_idx = pb[0, sl]
            if gat:
                d_idx = pb[1, sl]
                a = (plsc.load_gather(al_v, [s_idx])
                     + plsc.load_gather(ar_v, [d_idx])
                     + plsc.bitcast(pb[2, sl], jnp.float32))
                a = jnp.where(a >= 0.0, a, 0.2 * a)
                w = jnp.exp(a)
                plsc.addupdate_scatter(den_v, [d_idx], w)
            else:
                w = plsc.load_gather(al_v, [s_idx])
            w_v[sl] = w
        gdesc.wait()

        def scale(g, _):
            wvec = w_v[pl.ds(g * L, L)]
            for i in range(L):
                r = g * L + i
                wr = wvec[i]
                for j in range(D // L):
                    rows_v[r, pl.ds(j * L, L)] = rows_v[r, pl.ds(j * L, L)] * wr
            return 0

        lax.fori_loop(0, CHUNK // L, scale, 0)
        pltpu.sync_copy(rows_v, num_sh.at[pb.at[1]], add=True)
        pltpu.make_async_copy(pack_hbm.at[wid, c + 1], pbuf[b ^ 1],
                              sp[b ^ 1]).wait()

    def body(t, _):
        block(t, 0)
        block(t, 1)
        return 0

    lax.fori_loop(0, NCHUNK // 2, body, 0)
    plsc.subcore_barrier()
    for i in range(RPT // CHUNK):
        sl = pl.ds(sidx * RPT + i * CHUNK, CHUNK)
        pltpu.sync_copy(num_sh.at[sl], num_out.at[cidx, sl])
    if gat:
        pltpu.sync_copy(den_v, den_out.at[wid])


def _make_layer_call(gat):
    pb = pltpu.VMEM((3, CHUNK), jnp.int32)
    rows = pltpu.VMEM((CHUNK, D), jnp.float32)
    wv = pltpu.VMEM((CHUNK,), jnp.float32)
    tab = pltpu.VMEM((NPAD,), jnp.float32)
    sems = [pltpu.SemaphoreType.DMA] * 3
    if gat:
        out_type = [jax.ShapeDtypeStruct((2, NPAD, D), jnp.float32),
                    jax.ShapeDtypeStruct((NW, NPAD), jnp.float32)]
        scratch = [pb, pb, tab, tab, tab, wv, rows,
                   pltpu.VMEM_SHARED((NPAD, D), jnp.float32)] + sems
    else:
        out_type = [jax.ShapeDtypeStruct((2, NPAD, D), jnp.float32)]
        scratch = [pb, pb, tab, wv, rows,
                   pltpu.VMEM_SHARED((NPAD, D), jnp.float32)] + sems
    return pl.kernel(
        functools.partial(_layer_body, gat),
        out_type=out_type,
        mesh=_MESH,
        scratch_types=scratch,
        compiler_params=_SC_PARAMS,
    )


_gcn_call = _make_layer_call(False)
_gat_call = _make_layer_call(True)


def kernel(x, edge_index, edge_attr, batch, W_gcn, b_gcn, W_gat1, att_src1,
           att_dst1, We1, att_e1, b_gat1, W_gat2, att_src2, att_dst2, We2,
           att_e2, b_gat2, W_fc1, b_fc1, W_fc2, b_fc2, W_g1, b_g1, W_g2,
           b_g2, mol_bias):
    src = edge_index[0]
    dst = edge_index[1]
    pad = EPAD - E
    padi = jnp.full((pad,), N, jnp.int32)
    src_p = jnp.concatenate([src, padi])
    dst_p = jnp.concatenate([dst, padi])
    dst3 = dst_p.reshape(NW, NCHUNK, CHUNK)
    es1 = edge_attr @ (We1 @ att_e1)
    es2 = edge_attr @ (We2 @ att_e2)
    padf = jnp.zeros((pad,), jnp.float32)
    es1_p = jnp.concatenate([es1, padf])
    es2_p = jnp.concatenate([es2, padf])
    es1_3 = es1_p.reshape(NW, NCHUNK, CHUNK)
    es2_3 = es2_p.reshape(NW, NCHUNK, CHUNK)

    def mk_pack(es_bits):
        arr = jnp.stack([src_p, dst_p, es_bits], axis=0)
        arr = arr.reshape(3, NW, NCHUNK, CHUNK).transpose(1, 2, 0, 3)
        tail = jnp.full((NW, 1, 3, CHUNK), N, jnp.int32)
        return jnp.concatenate([arr, tail], axis=1)

    pack1 = mk_pack(lax.bitcast_convert_type(es1_p, jnp.int32))
    pack2 = mk_pack(lax.bitcast_convert_type(es2_p, jnp.int32))

    cnt_p, s1_p, s2_p = _p0_call(dst3, es1_3, es2_3)
    cnt = jnp.sum(cnt_p, axis=0)[:N]
    mean1 = jnp.sum(s1_p, axis=0)[:N] / jnp.maximum(cnt, 1.0)
    mean2 = jnp.sum(s2_p, axis=0)[:N] / jnp.maximum(cnt, 1.0)
    dis = lax.rsqrt(cnt + 1.0)
    dis_pad = jnp.concatenate([dis, jnp.ones((NPAD - N,), jnp.float32)])

    x_pad = jnp.concatenate([x, jnp.zeros((NPAD - N, D), jnp.float32)])

    xw = _mm(x_pad, W_gcn)
    (num,) = _gcn_call(xw, pack1, dis_pad)
    num = (num[0] + num[1])[:N]
    h = jnp.maximum(dis[:, None] * num
                    + (dis * dis)[:, None] * xw[:N] + b_gcn, 0.0)

    def gat_layer(h, W, a_s, a_d, pack, mean_ae, b, relu):
        h_pad = jnp.concatenate([h, jnp.zeros((NPAD - N, D), jnp.float32)])
        xw = _mm(h_pad, W)
        al = xw @ a_s
        ar = xw @ a_d
        num, den_p = _gat_call(xw, pack, al, ar)
        a_loop = al[:N] + ar[:N] + mean_ae
        a_loop = jnp.where(a_loop >= 0.0, a_loop, 0.2 * a_loop)
        w_loop = jnp.exp(a_loop)
        num = (num[0] + num[1])[:N] + w_loop[:, None] * xw[:N]
        den = jnp.sum(den_p, axis=0)[:N] + w_loop
        out = num / den[:, None] + b
        return jnp.maximum(out, 0.0) if relu else out

    h = gat_layer(h, W_gat1, att_src1, att_dst1, pack1, mean1, b_gat1, True)
    h = gat_layer(h, W_gat2, att_src2, att_dst2, pack2, mean2, b_gat2, False)

    pooled = jax.ops.segment_max(h, batch, num_segments=G)
    pooled = jnp.where(jnp.isfinite(pooled), pooled, 0.0)
    g = jnp.maximum(pooled @ W_g1 + b_g1, 0.0)
    return g @ W_g2 + b_g2


# speedup vs baseline: 1.2403x; 1.2403x over previous
import functools

import jax
import jax.numpy as jnp
from jax import lax
from jax.experimental import pallas as pl
from jax.experimental.pallas import tpu as pltpu
from jax.experimental.pallas import tpu_sc as plsc


N = 10000
E = 320000
D = 128
G = 256

NPAD = 10240
NW = 32
CHUNK = 128
NCHUNK = 79
EPW = CHUNK * NCHUNK
EPAD = NW * EPW
RPT = NPAD // 16
L = 16


def _mm_kernel(x_ref, w_ref, o_ref):
    o_ref[...] = jnp.dot(x_ref[...], w_ref[...],
                         preferred_element_type=jnp.float32)


def _mm(x, w, block=1024):
    m, k = x.shape
    n = w.shape[1]
    return pl.pallas_call(
        _mm_kernel,
        grid=(m // block,),
        in_specs=[
            pl.BlockSpec((block, k), lambda i: (i, 0)),
            pl.BlockSpec((k, n), lambda i: (0, 0)),
        ],
        out_specs=pl.BlockSpec((block, n), lambda i: (i, 0)),
        out_shape=jax.ShapeDtypeStruct((m, n), jnp.float32),
    )(x, w)


def _zero_1d(ref):
    z = jnp.zeros((L,), jnp.float32)

    def body(i, _):
        ref[pl.ds(i * L, L)] = z
        return 0

    lax.fori_loop(0, ref.shape[0] // L, body, 0)


def _zero_rows(ref):
    z = jnp.zeros((L,), jnp.float32)

    def body(r, _):
        for j in range(D // L):
            ref[r, pl.ds(j * L, L)] = z
        return 0

    lax.fori_loop(0, ref.shape[0], body, 0)


_MESH = plsc.VectorSubcoreMesh(core_axis_name="c", subcore_axis_name="s")
_SC_PARAMS = pltpu.CompilerParams(needs_layout_passes=False)


def _p0_body(dst_hbm, e1_hbm, e2_hbm, cnt_out, s1_out, s2_out,
             dst_v, e1_v, e2_v, cnt_v, s1_v, s2_v):
    cidx = lax.axis_index("c")
    sidx = lax.axis_index("s")
    wid = cidx * 16 + sidx
    pltpu.sync_copy(dst_hbm.at[wid], dst_v)
    pltpu.sync_copy(e1_hbm.at[wid], e1_v)
    pltpu.sync_copy(e2_hbm.at[wid], e2_v)
    _zero_1d(cnt_v)
    _zero_1d(s1_v)
    _zero_1d(s2_v)
    ones = jnp.ones((L,), jnp.float32)

    def body(c, _):
        for k in range(CHUNK // L):
            didx = dst_v[c, pl.ds(k * L, L)]
            plsc.addupdate_scatter(cnt_v, [didx], ones)
            plsc.addupdate_scatter(s1_v, [didx], e1_v[c, pl.ds(k * L, L)])
            plsc.addupdate_scatter(s2_v, [didx], e2_v[c, pl.ds(k * L, L)])
        return 0

    lax.fori_loop(0, NCHUNK, body, 0)
    pltpu.sync_copy(cnt_v, cnt_out.at[wid])
    pltpu.sync_copy(s1_v, s1_out.at[wid])
    pltpu.sync_copy(s2_v, s2_out.at[wid])


_p0_call = pl.kernel(
    _p0_body,
    out_type=[jax.ShapeDtypeStruct((NW, NPAD), jnp.float32)] * 3,
    mesh=_MESH,
    compiler_params=_SC_PARAMS,
    scratch_types=[
        pltpu.VMEM((NCHUNK, CHUNK), jnp.int32),
        pltpu.VMEM((NCHUNK, CHUNK), jnp.float32),
        pltpu.VMEM((NCHUNK, CHUNK), jnp.float32),
        pltpu.VMEM((NPAD,), jnp.float32),
        pltpu.VMEM((NPAD,), jnp.float32),
        pltpu.VMEM((NPAD,), jnp.float32),
    ],
)


def _layer_body(gat, *refs):
    if gat:
        (xw_hbm, pack_hbm, al_hbm, ar_hbm,
         num_out, den_out,
         pb, al_v, ar_v, den_v, w_v, rows_v, num_sh, sem) = refs
    else:
        (xw_hbm, pack_hbm, al_hbm,
         num_out,
         pb, al_v, w_v, rows_v, num_sh, sem) = refs
    cidx = lax.axis_index("c")
    sidx = lax.axis_index("s")
    wid = cidx * 16 + sidx

    pltpu.sync_copy(al_hbm, al_v)
    if gat:
        pltpu.sync_copy(ar_hbm, ar_v)
        _zero_1d(den_v)

    _zero_rows(rows_v)
    for i in range(RPT // CHUNK):
        pltpu.sync_copy(rows_v, num_sh.at[pl.ds(sidx * RPT + i * CHUNK, CHUNK)])
    plsc.subcore_barrier()

    def body(c, _):
        pltpu.sync_copy(pack_hbm.at[wid, c], pb)
        for k in range(CHUNK // L):
            sl = pl.ds(k * L, L)
            s_idx = pb[0, sl]
            if gat:
                d_idx = pb[1, sl]
                a = (plsc.load_gather(al_v, [s_idx])
                     + plsc.load_gather(ar_v, [d_idx])
                     + plsc.bitcast(pb[2, sl], jnp.float32))
                a = jnp.where(a >= 0.0, a, 0.2 * a)
                w = jnp.exp(a)
                plsc.addupdate_scatter(den_v, [d_idx], w)
            else:
                w = plsc.load_gather(al_v, [s_idx])
            w_v[sl] = w
        pltpu.async_copy(xw_hbm.at[pb.at[0]], rows_v, sem).wait()

        def scale(g, _):
            wvec = w_v[pl.ds(g * L, L)]
            for i in range(L):
                r = g * L + i
                wr = wvec[i]
                for j in range(D // L):
                    rows_v[r, pl.ds(j * L, L)] = rows_v[r, pl.ds(j * L, L)] * wr
            return 0

        lax.fori_loop(0, CHUNK // L, scale, 0)
        pltpu.sync_copy(rows_v, num_sh.at[pb.at[1]], add=True)
        return 0

    lax.fori_loop(0, NCHUNK, body, 0)
    plsc.subcore_barrier()
    for i in range(RPT // CHUNK):
        sl = pl.ds(sidx * RPT + i * CHUNK, CHUNK)
        pltpu.sync_copy(num_sh.at[sl], num_out.at[cidx, sl])
    if gat:
        pltpu.sync_copy(den_v, den_out.at[wid])


def _make_layer_call(gat):
    pb = pltpu.VMEM((3, CHUNK), jnp.int32)
    rows = pltpu.VMEM((CHUNK, D), jnp.float32)
    wv = pltpu.VMEM((CHUNK,), jnp.float32)
    tab = pltpu.VMEM((NPAD,), jnp.float32)
    if gat:
        out_type = [jax.ShapeDtypeStruct((2, NPAD, D), jnp.float32),
                    jax.ShapeDtypeStruct((NW, NPAD), jnp.float32)]
        scratch = [pb, tab, tab, tab, wv, rows,
                   pltpu.VMEM_SHARED((NPAD, D), jnp.float32),
                   pltpu.SemaphoreType.DMA]
    else:
        out_type = [jax.ShapeDtypeStruct((2, NPAD, D), jnp.float32)]
        scratch = [pb, tab, wv, rows,
                   pltpu.VMEM_SHARED((NPAD, D), jnp.float32),
                   pltpu.SemaphoreType.DMA]
    return pl.kernel(
        functools.partial(_layer_body, gat),
        out_type=out_type,
        mesh=_MESH,
        scratch_types=scratch,
        compiler_params=_SC_PARAMS,
    )


_gcn_call = _make_layer_call(False)
_gat_call = _make_layer_call(True)


def kernel(x, edge_index, edge_attr, batch, W_gcn, b_gcn, W_gat1, att_src1,
           att_dst1, We1, att_e1, b_gat1, W_gat2, att_src2, att_dst2, We2,
           att_e2, b_gat2, W_fc1, b_fc1, W_fc2, b_fc2, W_g1, b_g1, W_g2,
           b_g2, mol_bias):
    src = edge_index[0]
    dst = edge_index[1]
    pad = EPAD - E
    padi = jnp.full((pad,), N, jnp.int32)
    src_p = jnp.concatenate([src, padi])
    dst_p = jnp.concatenate([dst, padi])
    dst3 = dst_p.reshape(NW, NCHUNK, CHUNK)
    es1 = edge_attr @ (We1 @ att_e1)
    es2 = edge_attr @ (We2 @ att_e2)
    padf = jnp.zeros((pad,), jnp.float32)
    es1_p = jnp.concatenate([es1, padf])
    es2_p = jnp.concatenate([es2, padf])
    es1_3 = es1_p.reshape(NW, NCHUNK, CHUNK)
    es2_3 = es2_p.reshape(NW, NCHUNK, CHUNK)

    def mk_pack(es_bits):
        arr = jnp.stack([src_p, dst_p, es_bits], axis=0)
        arr = arr.reshape(3, NW, NCHUNK, CHUNK).transpose(1, 2, 0, 3)
        return arr

    pack1 = mk_pack(lax.bitcast_convert_type(es1_p, jnp.int32))
    pack2 = mk_pack(lax.bitcast_convert_type(es2_p, jnp.int32))

    cnt_p, s1_p, s2_p = _p0_call(dst3, es1_3, es2_3)
    cnt = jnp.sum(cnt_p, axis=0)[:N]
    mean1 = jnp.sum(s1_p, axis=0)[:N] / jnp.maximum(cnt, 1.0)
    mean2 = jnp.sum(s2_p, axis=0)[:N] / jnp.maximum(cnt, 1.0)
    dis = lax.rsqrt(cnt + 1.0)
    dis_pad = jnp.concatenate([dis, jnp.ones((NPAD - N,), jnp.float32)])

    x_pad = jnp.concatenate([x, jnp.zeros((NPAD - N, D), jnp.float32)])

    xw = _mm(x_pad, W_gcn)
    (num,) = _gcn_call(xw, pack1, dis_pad)
    num = (num[0] + num[1])[:N]
    h = jnp.maximum(dis[:, None] * num
                    + (dis * dis)[:, None] * xw[:N] + b_gcn, 0.0)

    def gat_layer(h, W, a_s, a_d, pack, mean_ae, b, relu):
        h_pad = jnp.concatenate([h, jnp.zeros((NPAD - N, D), jnp.float32)])
        xw = _mm(h_pad, W)
        al = xw @ a_s
        ar = xw @ a_d
        num, den_p = _gat_call(xw, pack, al, ar)
        a_loop = al[:N] + ar[:N] + mean_ae
        a_loop = jnp.where(a_loop >= 0.0, a_loop, 0.2 * a_loop)
        w_loop = jnp.exp(a_loop)
        num = (num[0] + num[1])[:N] + w_loop[:, None] * xw[:N]
        den = jnp.sum(den_p, axis=0)[:N] + w_loop
        out = num / den[:, None] + b
        return jnp.maximum(out, 0.0) if relu else out

    h = gat_layer(h, W_gat1, att_src1, att_dst1, pack1, mean1, b_gat1, True)
    h = gat_layer(h, W_gat2, att_src2, att_dst2, pack2, mean2, b_gat2, False)

    pooled = jax.ops.segment_max(h, batch, num_segments=G)
    pooled = jnp.where(jnp.isfinite(pooled), pooled, 0.0)
    g = jnp.maximum(pooled @ W_g1 + b_g1, 0.0)
    return g @ W_g2 + b_g2


# speedup vs baseline: 1.2417x; 1.0012x over previous
import functools

import jax
import jax.numpy as jnp
from jax import lax
from jax.experimental import pallas as pl
from jax.experimental.pallas import tpu as pltpu
from jax.experimental.pallas import tpu_sc as plsc


N = 10000
E = 320000
D = 128
G = 256

NPAD = 10240
NW = 32
CHUNK = 128
NCHUNK = 79
EPW = CHUNK * NCHUNK
EPAD = NW * EPW
RPT = NPAD // 16
L = 16


def _mm_kernel(x_ref, w_ref, o_ref):
    o_ref[...] = jnp.dot(x_ref[...], w_ref[...],
                         preferred_element_type=jnp.float32)


def _mm(x, w, block=1024):
    m, k = x.shape
    n = w.shape[1]
    return pl.pallas_call(
        _mm_kernel,
        grid=(m // block,),
        in_specs=[
            pl.BlockSpec((block, k), lambda i: (i, 0)),
            pl.BlockSpec((k, n), lambda i: (0, 0)),
        ],
        out_specs=pl.BlockSpec((block, n), lambda i: (i, 0)),
        out_shape=jax.ShapeDtypeStruct((m, n), jnp.float32),
    )(x, w)


def _zero_1d(ref):
    z = jnp.zeros((L,), jnp.float32)

    def body(i, _):
        ref[pl.ds(i * L, L)] = z
        return 0

    lax.fori_loop(0, ref.shape[0] // L, body, 0)


def _zero_rows(ref):
    z = jnp.zeros((L,), jnp.float32)

    def body(r, _):
        for j in range(D // L):
            ref[r, pl.ds(j * L, L)] = z
        return 0

    lax.fori_loop(0, ref.shape[0], body, 0)


_MESH = plsc.VectorSubcoreMesh(core_axis_name="c", subcore_axis_name="s")
_SC_PARAMS = pltpu.CompilerParams(needs_layout_passes=False)


def _p0_body(dst_hbm, e1_hbm, e2_hbm, cnt_out, s1_out, s2_out,
             dst_v, e1_v, e2_v, cnt_v, s1_v, s2_v):
    cidx = lax.axis_index("c")
    sidx = lax.axis_index("s")
    wid = cidx * 16 + sidx
    pltpu.sync_copy(dst_hbm.at[wid], dst_v)
    pltpu.sync_copy(e1_hbm.at[wid], e1_v)
    pltpu.sync_copy(e2_hbm.at[wid], e2_v)
    _zero_1d(cnt_v)
    _zero_1d(s1_v)
    _zero_1d(s2_v)
    ones = jnp.ones((L,), jnp.float32)

    def body(c, _):
        for k in range(CHUNK // L):
            didx = dst_v[c, pl.ds(k * L, L)]
            plsc.addupdate_scatter(cnt_v, [didx], ones)
            plsc.addupdate_scatter(s1_v, [didx], e1_v[c, pl.ds(k * L, L)])
            plsc.addupdate_scatter(s2_v, [didx], e2_v[c, pl.ds(k * L, L)])
        return 0

    lax.fori_loop(0, NCHUNK, body, 0)
    pltpu.sync_copy(cnt_v, cnt_out.at[wid])
    pltpu.sync_copy(s1_v, s1_out.at[wid])
    pltpu.sync_copy(s2_v, s2_out.at[wid])


_p0_call = pl.kernel(
    _p0_body,
    out_type=[jax.ShapeDtypeStruct((NW, NPAD), jnp.float32)] * 3,
    mesh=_MESH,
    compiler_params=_SC_PARAMS,
    scratch_types=[
        pltpu.VMEM((NCHUNK, CHUNK), jnp.int32),
        pltpu.VMEM((NCHUNK, CHUNK), jnp.float32),
        pltpu.VMEM((NCHUNK, CHUNK), jnp.float32),
        pltpu.VMEM((NPAD,), jnp.float32),
        pltpu.VMEM((NPAD,), jnp.float32),
        pltpu.VMEM((NPAD,), jnp.float32),
    ],
)


def _layer_body(gat, *refs):
    if gat:
        (xw_hbm, pack_hbm, al_hbm, ar_hbm,
         num_out, den_out,
         pb, al_v, ar_v, den_v, w_v, rows_v, num_sh, sem) = refs
    else:
        (xw_hbm, pack_hbm, al_hbm,
         num_out,
         pb, al_v, w_v, rows_v, num_sh, sem) = refs
    cidx = lax.axis_index("c")
    sidx = lax.axis_index("s")
    wid = cidx * 16 + sidx

    pltpu.sync_copy(al_hbm, al_v)
    if gat:
        pltpu.sync_copy(ar_hbm, ar_v)
        _zero_1d(den_v)

    _zero_rows(rows_v)
    for i in range(RPT // CHUNK):
        pltpu.async_copy(rows_v, num_sh.at[pl.ds(sidx * RPT + i * CHUNK, CHUNK)],
                         sem)
    for i in range(RPT // CHUNK):
        pltpu.make_async_copy(rows_v,
                              num_sh.at[pl.ds(sidx * RPT + i * CHUNK, CHUNK)],
                              sem).wait()
    plsc.subcore_barrier()

    def body(c, _):
        pltpu.sync_copy(pack_hbm.at[wid, c], pb)
        for k in range(CHUNK // L):
            sl = pl.ds(k * L, L)
            s_idx = pb[0, sl]
            if gat:
                d_idx = pb[1, sl]
                a = (plsc.load_gather(al_v, [s_idx])
                     + plsc.load_gather(ar_v, [d_idx])
                     + plsc.bitcast(pb[2, sl], jnp.float32))
                a = jnp.where(a >= 0.0, a, 0.2 * a)
                w = jnp.exp(a)
                plsc.addupdate_scatter(den_v, [d_idx], w)
            else:
                w = plsc.load_gather(al_v, [s_idx])
            w_v[sl] = w
        pltpu.async_copy(xw_hbm.at[pb.at[0]], rows_v, sem).wait()

        def scale(g, _):
            wvec = w_v[pl.ds(g * L, L)]
            for i in range(L):
                r = g * L + i
                wr = wvec[i]
                for j in range(D // L):
                    rows_v[r, pl.ds(j * L, L)] = rows_v[r, pl.ds(j * L, L)] * wr
            return 0

        lax.fori_loop(0, CHUNK // L, scale, 0)
        pltpu.sync_copy(rows_v, num_sh.at[pb.at[1]], add=True)
        return 0

    lax.fori_loop(0, NCHUNK, body, 0)
    plsc.subcore_barrier()
    for i in range(RPT // CHUNK):
        sl = pl.ds(sidx * RPT + i * CHUNK, CHUNK)
        pltpu.async_copy(num_sh.at[sl], num_out.at[cidx, sl], sem)
    for i in range(RPT // CHUNK):
        sl = pl.ds(sidx * RPT + i * CHUNK, CHUNK)
        pltpu.make_async_copy(num_sh.at[sl], num_out.at[cidx, sl], sem).wait()
    if gat:
        pltpu.sync_copy(den_v, den_out.at[wid])


def _make_layer_call(gat):
    pb = pltpu.VMEM((3, CHUNK), jnp.int32)
    rows = pltpu.VMEM((CHUNK, D), jnp.float32)
    wv = pltpu.VMEM((CHUNK,), jnp.float32)
    tab = pltpu.VMEM((NPAD,), jnp.float32)
    if gat:
        out_type = [jax.ShapeDtypeStruct((2, NPAD, D), jnp.float32),
                    jax.ShapeDtypeStruct((NW, NPAD), jnp.float32)]
        scratch = [pb, tab, tab, tab, wv, rows,
                   pltpu.VMEM_SHARED((NPAD, D), jnp.float32),
                   pltpu.SemaphoreType.DMA]
    else:
        out_type = [jax.ShapeDtypeStruct((2, NPAD, D), jnp.float32)]
        scratch = [pb, tab, wv, rows,
                   pltpu.VMEM_SHARED((NPAD, D), jnp.float32),
                   pltpu.SemaphoreType.DMA]
    return pl.kernel(
        functools.partial(_layer_body, gat),
        out_type=out_type,
        mesh=_MESH,
        scratch_types=scratch,
        compiler_params=_SC_PARAMS,
    )


_gcn_call = _make_layer_call(False)
_gat_call = _make_layer_call(True)


def kernel(x, edge_index, edge_attr, batch, W_gcn, b_gcn, W_gat1, att_src1,
           att_dst1, We1, att_e1, b_gat1, W_gat2, att_src2, att_dst2, We2,
           att_e2, b_gat2, W_fc1, b_fc1, W_fc2, b_fc2, W_g1, b_g1, W_g2,
           b_g2, mol_bias):
    src = edge_index[0]
    dst = edge_index[1]
    pad = EPAD - E
    padi = jnp.full((pad,), N, jnp.int32)
    src_p = jnp.concatenate([src, padi])
    dst_p = jnp.concatenate([dst, padi])
    dst3 = dst_p.reshape(NW, NCHUNK, CHUNK)
    es1 = edge_attr @ (We1 @ att_e1)
    es2 = edge_attr @ (We2 @ att_e2)
    padf = jnp.zeros((pad,), jnp.float32)
    es1_p = jnp.concatenate([es1, padf])
    es2_p = jnp.concatenate([es2, padf])
    es1_3 = es1_p.reshape(NW, NCHUNK, CHUNK)
    es2_3 = es2_p.reshape(NW, NCHUNK, CHUNK)

    def mk_pack(es_bits):
        arr = jnp.stack([src_p, dst_p, es_bits], axis=0)
        arr = arr.reshape(3, NW, NCHUNK, CHUNK).transpose(1, 2, 0, 3)
        return arr

    pack1 = mk_pack(lax.bitcast_convert_type(es1_p, jnp.int32))
    pack2 = mk_pack(lax.bitcast_convert_type(es2_p, jnp.int32))

    cnt_p, s1_p, s2_p = _p0_call(dst3, es1_3, es2_3)
    cnt = jnp.sum(cnt_p, axis=0)[:N]
    mean1 = jnp.sum(s1_p, axis=0)[:N] / jnp.maximum(cnt, 1.0)
    mean2 = jnp.sum(s2_p, axis=0)[:N] / jnp.maximum(cnt, 1.0)
    dis = lax.rsqrt(cnt + 1.0)
    dis_pad = jnp.concatenate([dis, jnp.ones((NPAD - N,), jnp.float32)])

    x_pad = jnp.concatenate([x, jnp.zeros((NPAD - N, D), jnp.float32)])

    xw = _mm(x_pad, W_gcn)
    (num,) = _gcn_call(xw, pack1, dis_pad)
    num = (num[0] + num[1])[:N]
    h = jnp.maximum(dis[:, None] * num
                    + (dis * dis)[:, None] * xw[:N] + b_gcn, 0.0)

    def gat_layer(h, W, a_s, a_d, pack, mean_ae, b, relu):
        h_pad = jnp.concatenate([h, jnp.zeros((NPAD - N, D), jnp.float32)])
        xw = _mm(h_pad, W)
        al = xw @ a_s
        ar = xw @ a_d
        num, den_p = _gat_call(xw, pack, al, ar)
        a_loop = al[:N] + ar[:N] + mean_ae
        a_loop = jnp.where(a_loop >= 0.0, a_loop, 0.2 * a_loop)
        w_loop = jnp.exp(a_loop)
        num = (num[0] + num[1])[:N] + w_loop[:, None] * xw[:N]
        den = jnp.sum(den_p, axis=0)[:N] + w_loop
        out = num / den[:, None] + b
        return jnp.maximum(out, 0.0) if relu else out

    h = gat_layer(h, W_gat1, att_src1, att_dst1, pack1, mean1, b_gat1, True)
    h = gat_layer(h, W_gat2, att_src2, att_dst2, pack2, mean2, b_gat2, False)

    pooled = jax.ops.segment_max(h, batch, num_segments=G)
    pooled = jnp.where(jnp.isfinite(pooled), pooled, 0.0)
    g = jnp.maximum(pooled @ W_g1 + b_g1, 0.0)
    return g @ W_g2 + b_g2
